# baseline (speedup 1.0000x reference)
import functools
import math

import jax
import jax.numpy as jnp
import numpy as np
from jax import lax
from jax.experimental import pallas as pl
from jax.experimental.pallas import tpu as pltpu
from jax.experimental.pallas import tpu_sc as plsc

_D = 128
_SEQ = 200
_CHUNK = 128
_NC, _NS = 2, 16
_NW = _NC * _NS


def _pe_np(d_model: int, seq: int) -> np.ndarray:
    pos = np.arange(seq, dtype=np.float32)[:, None]
    div = np.exp(np.arange(0, d_model, 2, dtype=np.float32)
                 * (-math.log(10000.0) / d_model))
    pe = np.zeros((seq, d_model), dtype=np.float32)
    pe[:, 0::2] = np.sin(pos * div)
    pe[:, 1::2] = np.cos(pos * div)
    return pe


_PE = _pe_np(_D, _SEQ)


def kernel(token_ids, table):
    B, L = token_ids.shape
    V, D = table.shape
    N = B * L
    n_per_w = N // _NW
    n_chunks = n_per_w // _CHUNK

    idx_flat = token_ids.reshape(N)
    pe = jnp.asarray(_PE)

    mesh = plsc.VectorSubcoreMesh(core_axis_name="c", subcore_axis_name="s")

    @functools.partial(
        pl.kernel,
        mesh=mesh,
        out_type=jax.ShapeDtypeStruct((N, D), jnp.float32),
        scratch_types=[
            pltpu.VMEM((_CHUNK,), jnp.int32),
            pltpu.VMEM((_CHUNK, _D), jnp.float32),
            pltpu.VMEM((_SEQ, _D), jnp.float32),
            pltpu.SemaphoreType.DMA,
        ],
    )
    def _emb(idx_hbm, pe_hbm, table_hbm, out_hbm, idx_v, rows_v, pe_v, sem):
        wid = lax.axis_index("s") * _NC + lax.axis_index("c")
        base = wid * n_per_w
        pltpu.sync_copy(pe_hbm, pe_v)

        def chunk_body(g, carry):
            start = base + g * _CHUNK
            pltpu.sync_copy(idx_hbm.at[pl.ds(start, _CHUNK)], idx_v)
            pltpu.async_copy(table_hbm.at[idx_v], rows_v, sem).wait()
            off = lax.rem(start, _SEQ)

            def row_body(r, c):
                pr = lax.rem(off + r, _SEQ)
                for cc in range(_D // 16):
                    sl = pl.ds(cc * 16, 16)
                    rows_v[r, sl] = rows_v[r, sl] + pe_v[pr, sl]
                return c

            lax.fori_loop(0, _CHUNK, row_body, 0)
            pltpu.sync_copy(rows_v, out_hbm.at[pl.ds(start, _CHUNK)])
            return carry

        lax.fori_loop(0, n_chunks, chunk_body, 0)

    out = _emb(idx_flat, pe, table)
    return out.reshape(B, L, D)


# speedup vs baseline: 1.5058x; 1.5058x over previous
import functools
import math

import jax
import jax.numpy as jnp
import numpy as np
from jax import lax
from jax.experimental import pallas as pl
from jax.experimental.pallas import tpu as pltpu
from jax.experimental.pallas import tpu_sc as plsc

_D = 128
_SEQ = 200
_CHUNK = 128
_NC, _NS = 2, 16
_NW = _NC * _NS


def _pe_np(d_model: int, seq: int) -> np.ndarray:
    pos = np.arange(seq, dtype=np.float32)[:, None]
    div = np.exp(np.arange(0, d_model, 2, dtype=np.float32)
                 * (-math.log(10000.0) / d_model))
    pe = np.zeros((seq, d_model), dtype=np.float32)
    pe[:, 0::2] = np.sin(pos * div)
    pe[:, 1::2] = np.cos(pos * div)
    return pe


_PE = _pe_np(_D, _SEQ)


def kernel(token_ids, table):
    B, L = token_ids.shape
    V, D = table.shape
    N = B * L
    n_per_w = N // _NW
    n_chunks = n_per_w // _CHUNK

    idx_flat = token_ids.reshape(N)
    pe = jnp.asarray(_PE)

    mesh = plsc.VectorSubcoreMesh(core_axis_name="c", subcore_axis_name="s")

    @functools.partial(
        pl.kernel,
        mesh=mesh,
        out_type=jax.ShapeDtypeStruct((N, D), jnp.float32),
        scratch_types=[
            pltpu.VMEM((n_per_w,), jnp.int32),
            pltpu.VMEM((_CHUNK, _D), jnp.float32),
            pltpu.VMEM((_CHUNK, _D), jnp.float32),
            pltpu.VMEM((_CHUNK, _D), jnp.float32),
            pltpu.VMEM((_CHUNK, _D), jnp.float32),
            pltpu.VMEM((_SEQ, _D), jnp.float32),
            pltpu.SemaphoreType.DMA,
            pltpu.SemaphoreType.DMA,
            pltpu.SemaphoreType.DMA,
            pltpu.SemaphoreType.DMA,
        ],
    )
    def _emb(idx_hbm, pe_hbm, table_hbm, out_hbm,
             idx_v, rowsa, rowsb, oba, obb, pe_v, ga, gb, oa, ob):
        wid = lax.axis_index("s") * _NC + lax.axis_index("c")
        base = wid * n_per_w
        pltpu.sync_copy(idx_hbm.at[pl.ds(base, n_per_w)], idx_v)
        pltpu.sync_copy(pe_hbm, pe_v)

        rows = (rowsa, rowsb)
        obuf = (oba, obb)
        gsem = (ga, gb)
        osem = (oa, ob)

        def g_desc(c, b):
            return pltpu.make_async_copy(
                table_hbm.at[idx_v.at[pl.ds(c * _CHUNK, _CHUNK)]],
                rows[b], gsem[b])

        def o_desc(c, b):
            return pltpu.make_async_copy(
                obuf[b], out_hbm.at[pl.ds(base + c * _CHUNK, _CHUNK)], osem[b])

        g_desc(0, 0).start()
        g_desc(1, 1).start()

        def outer(i, carry):
            c0 = i * 2
            for b in range(2):
                c = c0 + b
                g_desc(c, b).wait()

                @pl.when(c >= 2)
                def _():
                    o_desc(c - 2, b).wait()

                off = lax.rem(c * _CHUNK, _SEQ)

                def row_body(r, cr):
                    pr = lax.rem(off + r, _SEQ)
                    for cc in range(_D // 16):
                        sl = pl.ds(cc * 16, 16)
                        obuf[b][r, sl] = rows[b][r, sl] + pe_v[pr, sl]
                    return cr

                lax.fori_loop(0, _CHUNK, row_body, 0)

                @pl.when(c + 2 < n_chunks)
                def _():
                    g_desc(c + 2, b).start()

                o_desc(c, b).start()
            return carry

        lax.fori_loop(0, n_chunks // 2, outer, 0)
        o_desc(n_chunks - 2, 0).wait()
        o_desc(n_chunks - 1, 1).wait()

    out = _emb(idx_flat, pe, table)
    return out.reshape(B, L, D)


# speedup vs baseline: 3.7604x; 2.4973x over previous
import functools
import math

import jax
import jax.numpy as jnp
import numpy as np
from jax import lax
from jax.experimental import pallas as pl
from jax.experimental.pallas import tpu as pltpu
from jax.experimental.pallas import tpu_sc as plsc

_D = 128
_SEQ = 200
_CHUNK = 128
_NC, _NS = 2, 16
_NW = _NC * _NS
_PPC = _CHUNK // 32


def _pe_np(d_model: int, seq: int) -> np.ndarray:
    pos = np.arange(seq, dtype=np.float32)[:, None]
    div = np.exp(np.arange(0, d_model, 2, dtype=np.float32)
                 * (-math.log(10000.0) / d_model))
    pe = np.zeros((seq, d_model), dtype=np.float32)
    pe[:, 0::2] = np.sin(pos * div)
    pe[:, 1::2] = np.cos(pos * div)
    return pe


_PE = _pe_np(_D, _SEQ)


def _oidx_np(n_per_w: int, n_seq_w: int, seq: int) -> np.ndarray:
    w = np.arange(_NW, dtype=np.int32)[:, None, None]
    p = np.arange(seq, dtype=np.int32)[None, :, None]
    b = np.arange(n_seq_w, dtype=np.int32)[None, None, :]
    full = w * n_per_w + b * seq + p
    return full.reshape(_NW, (seq * n_seq_w) // _CHUNK, _CHUNK)


def kernel(token_ids, table):
    B, L = token_ids.shape
    V, D = table.shape
    N = B * L
    n_per_w = N // _NW
    n_chunks = n_per_w // _CHUNK
    n_seq_w = n_per_w // _SEQ

    perm_ids = token_ids.reshape(_NW, n_seq_w, L).transpose(0, 2, 1).reshape(N)
    pe = jnp.asarray(_PE)
    oidx = jnp.asarray(_oidx_np(n_per_w, n_seq_w, L))

    mesh = plsc.VectorSubcoreMesh(core_axis_name="c", subcore_axis_name="s")

    @functools.partial(
        pl.kernel,
        mesh=mesh,
        out_type=jax.ShapeDtypeStruct((N, D), jnp.float32),
        scratch_types=[
            pltpu.VMEM((n_per_w,), jnp.int32),
            pltpu.VMEM((n_chunks, _CHUNK), jnp.int32),
            pltpu.VMEM((_CHUNK, _D), jnp.float32),
            pltpu.VMEM((_CHUNK, _D), jnp.float32),
            pltpu.VMEM((_CHUNK, _D), jnp.float32),
            pltpu.VMEM((_CHUNK, _D), jnp.float32),
            pltpu.VMEM((_SEQ, _D), jnp.float32),
            pltpu.SemaphoreType.DMA,
            pltpu.SemaphoreType.DMA,
            pltpu.SemaphoreType.DMA,
            pltpu.SemaphoreType.DMA,
        ],
    )
    def _emb(pidx_hbm, oidx_hbm, pe_hbm, table_hbm, out_hbm,
             pidx_v, oidx_v, rowsa, rowsb, oba, obb, pe_v,
             ga, gb, oa, ob):
        wid = lax.axis_index("s") * _NC + lax.axis_index("c")
        base = wid * n_per_w
        pltpu.sync_copy(pidx_hbm.at[pl.ds(base, n_per_w)], pidx_v)
        pltpu.sync_copy(oidx_hbm.at[wid], oidx_v)
        pltpu.sync_copy(pe_hbm, pe_v)

        rows = (rowsa, rowsb)
        obuf = (oba, obb)
        gsem = (ga, gb)
        osem = (oa, ob)

        def g_desc(c, b):
            return pltpu.make_async_copy(
                table_hbm.at[pidx_v.at[pl.ds(c * _CHUNK, _CHUNK)]],
                rows[b], gsem[b])

        def o_desc(c, b):
            return pltpu.make_async_copy(
                obuf[b], out_hbm.at[oidx_v.at[c]], osem[b])

        g_desc(0, 0).start()
        g_desc(1, 1).start()

        def outer(i, carry):
            c0 = i * 2
            for b in range(2):
                c = c0 + b
                g_desc(c, b).wait()

                @pl.when(c >= 2)
                def _():
                    o_desc(c - 2, b).wait()

                for q in range(_PPC):
                    p = c * _PPC + q
                    pe_regs = [pe_v[p, pl.ds(k * 16, 16)] for k in range(8)]

                    def rbody(r, cr, _q=q, _pe=pe_regs):
                        row = _q * 32 + r * 2
                        for u in range(2):
                            for k in range(8):
                                sl = pl.ds(k * 16, 16)
                                obuf[b][row + u, sl] = (
                                    rows[b][row + u, sl] + _pe[k])
                        return cr

                    lax.fori_loop(0, 16, rbody, 0)

                @pl.when(c + 2 < n_chunks)
                def _():
                    g_desc(c + 2, b).start()

                o_desc(c, b).start()
            return carry

        lax.fori_loop(0, n_chunks // 2, outer, 0)
        o_desc(n_chunks - 2, 0).wait()
        o_desc(n_chunks - 1, 1).wait()

    out = _emb(perm_ids, oidx, pe, table)
    return out.reshape(B, L, D)
